# baseline (speedup 1.0000x reference)
import functools

import jax
import jax.numpy as jnp
from jax.experimental import pallas as pl
from jax.experimental.pallas import tpu as pltpu

KP_EXTENT = 0.15
B = 1280
S = 40
WB = 3200


def _wt_body(sT_ref, kp_ref, out_ref):
    sT = sT_ref[...]
    kp = kp_ref[...]
    sq = ((kp[:, 0:1] - sT[0:1, :]) ** 2
          + (kp[:, 1:2] - sT[1:2, :]) ** 2
          + (kp[:, 2:3] - sT[2:3, :]) ** 2)
    w = jnp.maximum(1.0 - jnp.sqrt(sq) / KP_EXTENT, 0.0)
    out_ref[...] = w.astype(jnp.bfloat16)


def _body(tt, bb, vv, wT_ref, inv_ref, x_ref, w_ref, out_ref, acc_ref,
          *, num_items, k_pts, c_in):
    i = pl.program_id(0)
    t = tt[i]
    prev_t = tt[jnp.maximum(i - 1, 0)]
    next_t = tt[jnp.minimum(i + 1, num_items - 1)]
    is_first = jnp.logical_or(i == 0, prev_t != t)
    is_last = jnp.logical_or(i == num_items - 1, next_t != t)

    @pl.when(is_first)
    def _():
        acc_ref[...] = jnp.zeros_like(acc_ref)

    wmat = wT_ref[...]
    inv = inv_ref[0]
    local = jnp.clip(inv - t * S, -1, S)
    local_bf = local.astype(jnp.bfloat16)
    iota_bf = jax.lax.broadcasted_iota(
        jnp.int32, (S, B), 0).astype(jnp.bfloat16)
    cmp = iota_bf == local_bf
    onehot = jnp.where(cmp, jnp.bfloat16(1), jnp.bfloat16(0))
    valid = (vv[i] > 0).astype(jnp.bfloat16)
    wmat_v = wmat * valid
    ohw = wmat_v[:, None, :] * onehot[None, :, :]
    z = jax.lax.dot_general(
        ohw.reshape(k_pts * S, B), x_ref[...],
        (((1,), (0,)), ((), ())), preferred_element_type=jnp.float32)
    acc_ref[...] += z.reshape(k_pts, S, c_in)

    @pl.when(is_last)
    def _():
        acc = acc_ref[...]
        o = jnp.zeros((S, out_ref.shape[-1]), jnp.float32)
        for k in range(k_pts):
            o = o + jnp.dot(acc[k].astype(jnp.bfloat16), w_ref[k],
                            preferred_element_type=jnp.float32)
        out_ref[...] = o


def kernel(s_pts, x, unq_inv, weights, kernel_points):
    n = x.shape[0]
    c_in = x.shape[1]
    k_pts = weights.shape[0]
    c_out = weights.shape[2]
    m = 10000
    nb = n // B
    nt = m // S
    max_items = nb + nt

    inv = unq_inv.astype(jnp.int32)
    sT = s_pts.T

    wT = pl.pallas_call(
        _wt_body,
        grid=(n // WB,),
        in_specs=[
            pl.BlockSpec((3, WB), lambda i: (0, i)),
            pl.BlockSpec((k_pts, 3), lambda i: (0, 0)),
        ],
        out_specs=pl.BlockSpec((k_pts, WB), lambda i: (0, i)),
        out_shape=jax.ShapeDtypeStruct((k_pts, n), jnp.bfloat16),
    )(sT, kernel_points)

    bounds = jnp.arange(nt + 1, dtype=jnp.int32) * S
    r = jnp.searchsorted(inv, bounds, side="left").astype(jnp.int32)
    r0, r1 = r[:-1], r[1:]
    blo = r0 // B
    bhi = (r1 + B - 1) // B
    nit = jnp.maximum(1, bhi - blo).astype(jnp.int32)
    ends = jnp.cumsum(nit)
    starts = ends - nit
    total = ends[-1]
    j = jnp.arange(max_items, dtype=jnp.int32)
    item_tile = jnp.minimum(
        jnp.searchsorted(ends, j, side="right").astype(jnp.int32), nt - 1)
    item_block = jnp.clip(blo[item_tile] + (j - starts[item_tile]), 0, nb - 1)
    item_valid = (j < total).astype(jnp.int32)

    inv3 = inv.reshape(nb, 1, B)
    x_bf = x.astype(jnp.bfloat16)
    weights_bf = weights.astype(jnp.bfloat16)

    grid_spec = pltpu.PrefetchScalarGridSpec(
        num_scalar_prefetch=3,
        grid=(max_items,),
        in_specs=[
            pl.BlockSpec((k_pts, B), lambda i, tt, bb, vv: (0, bb[i])),
            pl.BlockSpec((1, 1, B), lambda i, tt, bb, vv: (bb[i], 0, 0)),
            pl.BlockSpec((B, c_in), lambda i, tt, bb, vv: (bb[i], 0)),
            pl.BlockSpec((k_pts, c_in, c_out), lambda i, tt, bb, vv: (0, 0, 0)),
        ],
        out_specs=pl.BlockSpec((S, c_out), lambda i, tt, bb, vv: (tt[i], 0)),
        scratch_shapes=[pltpu.VMEM((k_pts, S, c_in), jnp.float32)],
    )
    body = functools.partial(_body, num_items=max_items, k_pts=k_pts, c_in=c_in)
    out = pl.pallas_call(
        body,
        grid_spec=grid_spec,
        out_shape=jax.ShapeDtypeStruct((m, c_out), jnp.float32),
    )(item_tile, item_block, item_valid, wT, inv3, x_bf, weights_bf)
    return out


# speedup vs baseline: 1.2919x; 1.2919x over previous
import functools

import jax
import jax.numpy as jnp
from jax.experimental import pallas as pl
from jax.experimental.pallas import tpu as pltpu

KP_EXTENT = 0.15
B = 1280
S = 80
WB = 3200


def _wt_body(sT_ref, kp_ref, out_ref):
    sT = sT_ref[...]
    kp = kp_ref[...]
    sq = ((kp[:, 0:1] - sT[0:1, :]) ** 2
          + (kp[:, 1:2] - sT[1:2, :]) ** 2
          + (kp[:, 2:3] - sT[2:3, :]) ** 2)
    w = jnp.maximum(1.0 - jnp.sqrt(sq) / KP_EXTENT, 0.0)
    out_ref[...] = w.astype(jnp.bfloat16)


def _body(tt, bb, vv, wT_ref, inv_ref, x_ref, w_ref, out_ref, acc_ref,
          *, num_items, k_pts, c_in):
    i = pl.program_id(0)
    t = tt[i]
    prev_t = tt[jnp.maximum(i - 1, 0)]
    next_t = tt[jnp.minimum(i + 1, num_items - 1)]
    is_first = jnp.logical_or(i == 0, prev_t != t)
    is_last = jnp.logical_or(i == num_items - 1, next_t != t)

    @pl.when(is_first)
    def _():
        acc_ref[...] = jnp.zeros_like(acc_ref)

    wmat = wT_ref[...]
    inv = inv_ref[0]
    local = jnp.clip(inv - t * S, -1, S)
    local_bf = local.astype(jnp.bfloat16)
    iota_bf = jax.lax.broadcasted_iota(
        jnp.int32, (S, B), 0).astype(jnp.bfloat16)
    cmp = iota_bf == local_bf
    onehot = jnp.where(cmp, jnp.bfloat16(1), jnp.bfloat16(0))
    valid = (vv[i] > 0).astype(jnp.bfloat16)
    wmat_v = wmat * valid
    ohw = wmat_v[:, None, :] * onehot[None, :, :]
    z = jax.lax.dot_general(
        ohw.reshape(k_pts * S, B), x_ref[...],
        (((1,), (0,)), ((), ())), preferred_element_type=jnp.float32)
    acc_ref[...] += z.reshape(k_pts, S, c_in)

    @pl.when(is_last)
    def _():
        acc = acc_ref[...]
        o = jnp.zeros((S, out_ref.shape[-1]), jnp.float32)
        for k in range(k_pts):
            o = o + jnp.dot(acc[k].astype(jnp.bfloat16), w_ref[k],
                            preferred_element_type=jnp.float32)
        out_ref[...] = o


def kernel(s_pts, x, unq_inv, weights, kernel_points):
    n = x.shape[0]
    c_in = x.shape[1]
    k_pts = weights.shape[0]
    c_out = weights.shape[2]
    m = 10000
    nb = n // B
    nt = m // S
    max_items = nb + nt

    inv = unq_inv.astype(jnp.int32)
    sT = s_pts.T

    wT = pl.pallas_call(
        _wt_body,
        grid=(n // WB,),
        in_specs=[
            pl.BlockSpec((3, WB), lambda i: (0, i)),
            pl.BlockSpec((k_pts, 3), lambda i: (0, 0)),
        ],
        out_specs=pl.BlockSpec((k_pts, WB), lambda i: (0, i)),
        out_shape=jax.ShapeDtypeStruct((k_pts, n), jnp.bfloat16),
    )(sT, kernel_points)

    bounds = jnp.arange(nt + 1, dtype=jnp.int32) * S
    r = jnp.searchsorted(inv, bounds, side="left").astype(jnp.int32)
    r0, r1 = r[:-1], r[1:]
    blo = r0 // B
    bhi = (r1 + B - 1) // B
    nit = jnp.maximum(1, bhi - blo).astype(jnp.int32)
    ends = jnp.cumsum(nit)
    starts = ends - nit
    total = ends[-1]
    j = jnp.arange(max_items, dtype=jnp.int32)
    item_tile = jnp.minimum(
        jnp.searchsorted(ends, j, side="right").astype(jnp.int32), nt - 1)
    item_block = jnp.clip(blo[item_tile] + (j - starts[item_tile]), 0, nb - 1)
    item_valid = (j < total).astype(jnp.int32)

    inv3 = inv.reshape(nb, 1, B)
    x_bf = x.astype(jnp.bfloat16)
    weights_bf = weights.astype(jnp.bfloat16)

    grid_spec = pltpu.PrefetchScalarGridSpec(
        num_scalar_prefetch=3,
        grid=(max_items,),
        in_specs=[
            pl.BlockSpec((k_pts, B), lambda i, tt, bb, vv: (0, bb[i])),
            pl.BlockSpec((1, 1, B), lambda i, tt, bb, vv: (bb[i], 0, 0)),
            pl.BlockSpec((B, c_in), lambda i, tt, bb, vv: (bb[i], 0)),
            pl.BlockSpec((k_pts, c_in, c_out), lambda i, tt, bb, vv: (0, 0, 0)),
        ],
        out_specs=pl.BlockSpec((S, c_out), lambda i, tt, bb, vv: (tt[i], 0)),
        scratch_shapes=[pltpu.VMEM((k_pts, S, c_in), jnp.float32)],
    )
    body = functools.partial(_body, num_items=max_items, k_pts=k_pts, c_in=c_in)
    out = pl.pallas_call(
        body,
        grid_spec=grid_spec,
        out_shape=jax.ShapeDtypeStruct((m, c_out), jnp.float32),
    )(item_tile, item_block, item_valid, wT, inv3, x_bf, weights_bf)
    return out


# speedup vs baseline: 1.4497x; 1.1221x over previous
import functools

import jax
import jax.numpy as jnp
from jax.experimental import pallas as pl
from jax.experimental.pallas import tpu as pltpu

KP_EXTENT = 0.15
B = 1280
S = 80
WB = 3200


def _wt_body(sT_ref, kp_ref, out_ref):
    sT = sT_ref[...]
    kp = kp_ref[...]
    sq = ((kp[:, 0:1] - sT[0:1, :]) ** 2
          + (kp[:, 1:2] - sT[1:2, :]) ** 2
          + (kp[:, 2:3] - sT[2:3, :]) ** 2)
    w = jnp.maximum(1.0 - jnp.sqrt(sq) / KP_EXTENT, 0.0)
    out_ref[...] = w.astype(jnp.bfloat16)


def _body(tt, bb, vv, wT_ref, inv_ref, x_ref, w_ref, out_ref, acc_ref,
          *, num_items, k_pts, c_in):
    i = pl.program_id(0)
    t = tt[i]
    prev_t = tt[jnp.maximum(i - 1, 0)]
    next_t = tt[jnp.minimum(i + 1, num_items - 1)]
    is_first = jnp.logical_or(i == 0, prev_t != t)
    is_last = jnp.logical_or(i == num_items - 1, next_t != t)

    @pl.when(is_first)
    def _():
        acc_ref[...] = jnp.zeros_like(acc_ref)

    wmat = wT_ref[...]
    inv = inv_ref[0]
    local = jnp.clip(inv - t * S, -1, S)
    local_bf = local.astype(jnp.bfloat16)
    iota_bf = jax.lax.broadcasted_iota(
        jnp.int32, (S, B), 0).astype(jnp.bfloat16)
    cmp = iota_bf == local_bf
    onehot = jnp.where(cmp, jnp.bfloat16(1), jnp.bfloat16(0))
    valid = (vv[i] > 0).astype(jnp.bfloat16)
    wmat_v = wmat * valid
    ohw = wmat_v[:, None, :] * onehot[None, :, :]
    z = jax.lax.dot_general(
        ohw.reshape(k_pts * S, B), x_ref[...],
        (((1,), (0,)), ((), ())), preferred_element_type=jnp.float32)
    acc_ref[...] += z.reshape(k_pts, S, c_in)

    @pl.when(is_last)
    def _():
        acc = acc_ref[...]
        o = jnp.zeros((S, out_ref.shape[-1]), jnp.float32)
        for k in range(k_pts):
            o = o + jnp.dot(acc[k].astype(jnp.bfloat16), w_ref[k],
                            preferred_element_type=jnp.float32)
        out_ref[...] = o


def kernel(s_pts, x, unq_inv, weights, kernel_points):
    n = x.shape[0]
    c_in = x.shape[1]
    k_pts = weights.shape[0]
    c_out = weights.shape[2]
    m = 10000
    nb = n // B
    nt = m // S
    max_items = nb + nt

    inv = unq_inv.astype(jnp.int32)
    sT = s_pts.T

    wT = pl.pallas_call(
        _wt_body,
        grid=(n // WB,),
        in_specs=[
            pl.BlockSpec((3, WB), lambda i: (0, i)),
            pl.BlockSpec((k_pts, 3), lambda i: (0, 0)),
        ],
        out_specs=pl.BlockSpec((k_pts, WB), lambda i: (0, i)),
        out_shape=jax.ShapeDtypeStruct((k_pts, n), jnp.bfloat16),
    )(sT, kernel_points)

    inv2 = inv.reshape(nb, B)
    t_first = inv2[:, 0] // S
    t_last = inv2[:, B - 1] // S
    t_start = t_first.at[0].set(0)
    t_next = jnp.concatenate(
        [t_first[1:], jnp.full((1,), nt, jnp.int32)])
    t_cov_end = jnp.maximum(t_last, t_next - 1)
    ntiles = t_cov_end - t_start + 1
    ends = jnp.cumsum(ntiles)
    starts = ends - ntiles
    total = ends[-1]
    j = jnp.arange(max_items, dtype=jnp.int32)
    item_block = jnp.minimum(
        jnp.searchsorted(ends, j, side="right").astype(jnp.int32), nb - 1)
    item_tile = jnp.minimum(t_start[item_block] + (j - starts[item_block]),
                            nt - 1)
    item_valid = (j < total).astype(jnp.int32)

    inv3 = inv.reshape(nb, 1, B)
    x_bf = x.astype(jnp.bfloat16)
    weights_bf = weights.astype(jnp.bfloat16)

    grid_spec = pltpu.PrefetchScalarGridSpec(
        num_scalar_prefetch=3,
        grid=(max_items,),
        in_specs=[
            pl.BlockSpec((k_pts, B), lambda i, tt, bb, vv: (0, bb[i])),
            pl.BlockSpec((1, 1, B), lambda i, tt, bb, vv: (bb[i], 0, 0)),
            pl.BlockSpec((B, c_in), lambda i, tt, bb, vv: (bb[i], 0)),
            pl.BlockSpec((k_pts, c_in, c_out), lambda i, tt, bb, vv: (0, 0, 0)),
        ],
        out_specs=pl.BlockSpec((S, c_out), lambda i, tt, bb, vv: (tt[i], 0)),
        scratch_shapes=[pltpu.VMEM((k_pts, S, c_in), jnp.float32)],
    )
    body = functools.partial(_body, num_items=max_items, k_pts=k_pts, c_in=c_in)
    out = pl.pallas_call(
        body,
        grid_spec=grid_spec,
        out_shape=jax.ShapeDtypeStruct((m, c_out), jnp.float32),
    )(item_tile, item_block, item_valid, wT, inv3, x_bf, weights_bf)
    return out
